# baseline (speedup 1.0000x reference)
import functools

import jax
import jax.numpy as jnp
from jax import lax
from jax.experimental import pallas as pl
from jax.experimental.pallas import tpu as pltpu
from jax.experimental.pallas import tpu_sc as plsc

_N = 10000
_D = 128
_E = 320000
_EL = 100000

_NT = 32
_CH = 128
_EC = 80
_EP = _NT * _EC * _CH
_ROWS_PER_TILE = 640
_NP = 16 * _ROWS_PER_TILE
_DC = 25
_ELP = _NT * _DC * _CH
_RING = 4

@functools.lru_cache(maxsize=None)
def _mesh():
    return plsc.VectorSubcoreMesh(core_axis_name="c", subcore_axis_name="s")


def _agg_body(with_count, *refs):
    if with_count:
        (x_hbm, src_hbm, dst_hbm, zr_hbm, zc_hbm, agg_out, cnt_out,
         sidx, didx, rows, ones, agg_sp, cnt_sp, *sems) = refs
    else:
        (x_hbm, src_hbm, dst_hbm, zr_hbm, agg_out,
         sidx, didx, rows, agg_sp, *sems) = refs
    semg = sems[0:2]
    semi = sems[2:4]

    cid = lax.axis_index("c")
    sid = lax.axis_index("s")
    wid = sid * 2 + cid
    rbase = sid * _ROWS_PER_TILE
    ibase = wid * _EC

    def idx_load(k, b):
        pltpu.async_copy(src_hbm.at[ibase + k], sidx.at[b], semi[b])
        pltpu.async_copy(dst_hbm.at[ibase + k], didx.at[b], semi[b])

    def idx_wait(k, b):
        pltpu.make_async_copy(src_hbm.at[ibase + k], sidx.at[b], semi[b]).wait()
        pltpu.make_async_copy(dst_hbm.at[ibase + k], didx.at[b], semi[b]).wait()

    def gather(k, b):
        pltpu.async_copy(x_hbm.at[sidx.at[b]], rows.at[b], semg[b])

    def gather_wait(k, b):
        pltpu.make_async_copy(x_hbm.at[sidx.at[b]], rows.at[b], semg[b]).wait()

    def scatter(k, b):
        pltpu.sync_copy(rows.at[b], agg_sp.at[didx.at[b]], add=True)
        if with_count:
            pltpu.sync_copy(ones, cnt_sp.at[didx.at[b]], add=True)

    idx_load(0, 0)
    idx_load(1, 1)
    if with_count:
        for i in range(_CH // 16):
            ones[pl.ds(i * 16, 16)] = jnp.ones((16,), jnp.float32)
    idx_wait(0, 0)
    gather(0, 0)
    pltpu.sync_copy(zr_hbm, agg_sp.at[pl.ds(rbase, _ROWS_PER_TILE)])
    if with_count:
        pltpu.sync_copy(zc_hbm, cnt_sp.at[pl.ds(rbase, _ROWS_PER_TILE)])
    plsc.subcore_barrier()

    def step(k, b, issue_idx, issue_gather):
        gather_wait(k, b)
        if issue_gather:
            idx_wait(k + 1, 1 - b)
            gather(k + 1, 1 - b)
        scatter(k, b)
        if issue_idx:
            idx_load(k + 2, b)

    def body(i, carry):
        step(2 * i, 0, True, True)
        step(2 * i + 1, 1, True, True)
        return carry

    lax.fori_loop(0, (_EC - 2) // 2, body, 0)
    step(_EC - 2, 0, False, True)
    step(_EC - 1, 1, False, False)

    plsc.subcore_barrier()
    pltpu.sync_copy(agg_sp.at[pl.ds(rbase, _ROWS_PER_TILE)],
                    agg_out.at[cid, pl.ds(rbase, _ROWS_PER_TILE)])
    if with_count:
        pltpu.sync_copy(cnt_sp.at[pl.ds(rbase, _ROWS_PER_TILE)],
                        cnt_out.at[cid, pl.ds(rbase, _ROWS_PER_TILE)])


@functools.lru_cache(maxsize=None)
def _agg_count_call():
    return functools.partial(
        pl.kernel,
        mesh=_mesh(),
        out_type=[jax.ShapeDtypeStruct((2, _NP, _D), jnp.float32),
                  jax.ShapeDtypeStruct((2, _NP), jnp.float32)],
        scratch_types=[
            pltpu.VMEM((2, _CH), jnp.int32),
            pltpu.VMEM((2, _CH), jnp.int32),
            pltpu.VMEM((2, _CH, _D), jnp.float32),
            pltpu.VMEM((_CH,), jnp.float32),
            pltpu.VMEM_SHARED((_NP, _D), jnp.float32),
            pltpu.VMEM_SHARED((_NP,), jnp.float32),
        ] + [pltpu.SemaphoreType.DMA] * 4,
    )(functools.partial(_agg_body, True))


@functools.lru_cache(maxsize=None)
def _agg_call():
    return functools.partial(
        pl.kernel,
        mesh=_mesh(),
        out_type=[jax.ShapeDtypeStruct((2, _NP, _D), jnp.float32)],
        scratch_types=[
            pltpu.VMEM((2, _CH), jnp.int32),
            pltpu.VMEM((2, _CH), jnp.int32),
            pltpu.VMEM((2, _CH, _D), jnp.float32),
            pltpu.VMEM_SHARED((_NP, _D), jnp.float32),
        ] + [pltpu.SemaphoreType.DMA] * 4,
    )(functools.partial(_agg_body, False))


_R = 2048


def _tc_layer_body(relu, want_norm, agg_ref, cnt_ref, h_ref, wl_ref, wr_ref,
                   b_ref, z_ref, *nz_ref):
    a = agg_ref[0] + agg_ref[1]
    c = cnt_ref[0] + cnt_ref[1]
    mean = a * (1.0 / jnp.maximum(c, 1.0))
    dn = (((1,), (1,)), ((), ()))
    z = (lax.dot_general(mean, wl_ref[...], dn,
                         precision=lax.Precision.HIGHEST,
                         preferred_element_type=jnp.float32)
         + lax.dot_general(h_ref[...], wr_ref[...], dn,
                           precision=lax.Precision.HIGHEST,
                           preferred_element_type=jnp.float32)
         + b_ref[...])
    if relu:
        z = jnp.maximum(z, 0.0)
    z_ref[...] = z
    if want_norm:
        nz_ref[0][...] = jnp.sqrt(jnp.sum(z * z, axis=1, keepdims=True))


def _tc_layer(aggp, cntp, hin, Wl, Wr, b, relu, want_norm):
    grid = (_NP // _R,)
    out_shape = [jax.ShapeDtypeStruct((_NP, _D), jnp.float32)]
    out_specs = [pl.BlockSpec((_R, _D), lambda i: (i, 0))]
    if want_norm:
        out_shape.append(jax.ShapeDtypeStruct((_NP, 1), jnp.float32))
        out_specs.append(pl.BlockSpec((_R, 1), lambda i: (i, 0)))
    return pl.pallas_call(
        functools.partial(_tc_layer_body, relu, want_norm),
        grid=grid,
        in_specs=[
            pl.BlockSpec((2, _R, _D), lambda i: (0, i, 0)),
            pl.BlockSpec((2, _R, 1), lambda i: (0, i, 0)),
            pl.BlockSpec((_R, _D), lambda i: (i, 0)),
            pl.BlockSpec((_D, _D), lambda i: (0, 0)),
            pl.BlockSpec((_D, _D), lambda i: (0, 0)),
            pl.BlockSpec((1, _D), lambda i: (0, 0)),
        ],
        out_specs=out_specs,
        out_shape=out_shape,
    )(aggp, cntp, hin, Wl, Wr, b)


def _gather_pairs_body(z_hbm, ia_hbm, ib_hbm, za_out, zb_out,
                       aidx, bidx, za, zb, *sems):
    semi = sems[0:2]
    semga = sems[2:4]
    semgb = sems[4:6]
    semwa = sems[6:8]
    semwb = sems[8:10]
    cid = lax.axis_index("c")
    sid = lax.axis_index("s")
    wid = sid * 2 + cid
    ibase = wid * _DC
    obase = wid * (_DC * _CH)

    def idx_load(k, b):
        pltpu.async_copy(ia_hbm.at[ibase + k], aidx.at[b], semi[b])
        pltpu.async_copy(ib_hbm.at[ibase + k], bidx.at[b], semi[b])

    def idx_wait(k, b):
        pltpu.make_async_copy(ia_hbm.at[ibase + k], aidx.at[b], semi[b]).wait()
        pltpu.make_async_copy(ib_hbm.at[ibase + k], bidx.at[b], semi[b]).wait()

    def gather(k, b):
        pltpu.async_copy(z_hbm.at[aidx.at[b]], za.at[b], semga[b])
        pltpu.async_copy(z_hbm.at[bidx.at[b]], zb.at[b], semgb[b])

    def gather_wait(k, b):
        pltpu.make_async_copy(z_hbm.at[aidx.at[b]], za.at[b], semga[b]).wait()
        pltpu.make_async_copy(z_hbm.at[bidx.at[b]], zb.at[b], semgb[b]).wait()

    def write(k, b):
        o = obase + k * _CH
        pltpu.async_copy(za.at[b], za_out.at[pl.ds(o, _CH)], semwa[b])
        pltpu.async_copy(zb.at[b], zb_out.at[pl.ds(o, _CH)], semwb[b])

    def write_wait(k, b):
        o = obase + k * _CH
        pltpu.make_async_copy(za.at[b], za_out.at[pl.ds(o, _CH)], semwa[b]).wait()
        pltpu.make_async_copy(zb.at[b], zb_out.at[pl.ds(o, _CH)], semwb[b]).wait()

    def step(k, b, wait_prev, issue_idx, issue_gather):
        gather_wait(k, b)
        write(k, b)
        if wait_prev:
            write_wait(k - 1, 1 - b)
        if issue_gather:
            idx_wait(k + 1, 1 - b)
            gather(k + 1, 1 - b)
        if issue_idx:
            idx_load(k + 2, b)

    idx_load(0, 0)
    idx_load(1, 1)
    idx_wait(0, 0)
    gather(0, 0)
    step(0, 0, False, True, True)

    def body(i, carry):
        k = 2 * i + 1
        step(k, 1, True, True, True)
        step(k + 1, 0, True, True, True)
        return carry

    lax.fori_loop(0, (_DC - 3) // 2, body, 0)
    step(_DC - 2, 1, True, False, True)
    step(_DC - 1, 0, True, False, False)
    write_wait(_DC - 1, 0)


@functools.lru_cache(maxsize=None)
def _gather_pairs_call():
    return functools.partial(
        pl.kernel,
        mesh=_mesh(),
        out_type=[jax.ShapeDtypeStruct((_ELP, _D), jnp.float32),
                  jax.ShapeDtypeStruct((_ELP, _D), jnp.float32)],
        scratch_types=[
            pltpu.VMEM((2, _CH), jnp.int32),
            pltpu.VMEM((2, _CH), jnp.int32),
            pltpu.VMEM((2, _CH, _D), jnp.float32),
            pltpu.VMEM((2, _CH, _D), jnp.float32),
        ] + [pltpu.SemaphoreType.DMA] * 10,
    )(_gather_pairs_body)


_RD = 2048


def _cosine_body(za_ref, zb_ref, o_ref):
    za = za_ref[...]
    zb = zb_ref[...]
    num = jnp.sum(za * zb, axis=1, keepdims=True)
    sa = jnp.sum(za * za, axis=1, keepdims=True)
    sb = jnp.sum(zb * zb, axis=1, keepdims=True)
    den = jnp.maximum(jnp.sqrt(sa) * jnp.sqrt(sb), 1e-8)
    o_ref[...] = num / den


def _cosine(za, zb):
    return pl.pallas_call(
        _cosine_body,
        grid=(_ELP // _RD,),
        in_specs=[pl.BlockSpec((_RD, _D), lambda i: (i, 0)),
                  pl.BlockSpec((_RD, _D), lambda i: (i, 0))],
        out_specs=pl.BlockSpec((_RD, 1), lambda i: (i, 0)),
        out_shape=jax.ShapeDtypeStruct((_ELP, 1), jnp.float32),
    )(za, zb)


def kernel(x, edge_index, edge_label_index, W1l, W1r, b1, W2l, W2r, b2):
    src = edge_index[0]
    dst = edge_index[1]
    srcp = jnp.concatenate([src, jnp.zeros((_EP - _E,), jnp.int32)])
    srcp = srcp.reshape(_NT * _EC, _CH)
    dstp = jnp.concatenate([dst, jnp.full((_EP - _E,), _N, jnp.int32)])
    dstp = dstp.reshape(_NT * _EC, _CH)
    xp = jnp.concatenate([x, jnp.zeros((_NP - _N, _D), jnp.float32)], axis=0)
    zr = jnp.zeros((_ROWS_PER_TILE, _D), jnp.float32)
    zc = jnp.zeros((_ROWS_PER_TILE,), jnp.float32)

    agg1, cnt = _agg_count_call()(xp, srcp, dstp, zr, zc)
    cnt3 = cnt.reshape(2, _NP, 1)
    h = _tc_layer(agg1, cnt3, xp, W1l, W1r, b1.reshape(1, _D),
                  relu=True, want_norm=False)[0]
    (agg2,) = _agg_call()(h, srcp, dstp, zr)
    (z,) = _tc_layer(agg2, cnt3, h, W2l, W2r, b2.reshape(1, _D),
                     relu=False, want_norm=False)

    ea = jnp.concatenate([edge_label_index[0],
                          jnp.zeros((_ELP - _EL,), jnp.int32)])
    ea = ea.reshape(_NT * _DC, _CH)
    eb = jnp.concatenate([edge_label_index[1],
                          jnp.zeros((_ELP - _EL,), jnp.int32)])
    eb = eb.reshape(_NT * _DC, _CH)
    za, zb = _gather_pairs_call()(z, ea, eb)
    out = _cosine(za, zb)
    return out.reshape(_ELP)[:_EL]


# speedup vs baseline: 1.5034x; 1.5034x over previous
import functools

import jax
import jax.numpy as jnp
from jax import lax
from jax.experimental import pallas as pl
from jax.experimental.pallas import tpu as pltpu
from jax.experimental.pallas import tpu_sc as plsc

_N = 10000
_D = 128
_E = 320000
_EL = 100000

_NT = 32
_CH = 128
_FAST_CID = 0
_ECF = 122
_ECS = 36
_EP = 16 * (_ECF + _ECS) * _CH
_ROWS_PER_TILE = 640
_NP = 16 * _ROWS_PER_TILE
_DCF = 37
_DCS = 13
_ELP = 16 * (_DCF + _DCS) * _CH

@functools.lru_cache(maxsize=None)
def _mesh():
    return plsc.VectorSubcoreMesh(core_axis_name="c", subcore_axis_name="s")


def _agg_body(with_count, *refs):
    if with_count:
        (x_hbm, src_hbm, dst_hbm, zr_hbm, zc_hbm, agg_out, cnt_out,
         sidx, didx, rows, ones, agg_sp, cnt_sp, *sems) = refs
    else:
        (x_hbm, src_hbm, dst_hbm, zr_hbm, agg_out,
         sidx, didx, rows, agg_sp, *sems) = refs
    semg = sems[0:2]
    semi = sems[2:4]

    cid = lax.axis_index("c")
    sid = lax.axis_index("s")
    rbase = sid * _ROWS_PER_TILE
    fast = cid == _FAST_CID
    ec = jnp.where(fast, _ECF, _ECS)
    ibase = jnp.where(fast, sid * _ECF, 16 * _ECF + sid * _ECS)

    def idx_load(k, b):
        pltpu.async_copy(src_hbm.at[ibase + k], sidx.at[b], semi[b])
        pltpu.async_copy(dst_hbm.at[ibase + k], didx.at[b], semi[b])

    def idx_wait(k, b):
        pltpu.make_async_copy(src_hbm.at[ibase + k], sidx.at[b], semi[b]).wait()
        pltpu.make_async_copy(dst_hbm.at[ibase + k], didx.at[b], semi[b]).wait()

    def gather(k, b):
        pltpu.async_copy(x_hbm.at[sidx.at[b]], rows.at[b], semg[b])

    def gather_wait(k, b):
        pltpu.make_async_copy(x_hbm.at[sidx.at[b]], rows.at[b], semg[b]).wait()

    def scatter(k, b):
        pltpu.sync_copy(rows.at[b], agg_sp.at[didx.at[b]], add=True)
        if with_count:
            pltpu.sync_copy(ones, cnt_sp.at[didx.at[b]], add=True)

    idx_load(0, 0)
    idx_load(1, 1)
    if with_count:
        for i in range(_CH // 16):
            ones[pl.ds(i * 16, 16)] = jnp.ones((16,), jnp.float32)
    idx_wait(0, 0)
    gather(0, 0)
    pltpu.sync_copy(zr_hbm, agg_sp.at[pl.ds(rbase, _ROWS_PER_TILE)])
    if with_count:
        pltpu.sync_copy(zc_hbm, cnt_sp.at[pl.ds(rbase, _ROWS_PER_TILE)])
    plsc.subcore_barrier()

    def step(k, b, issue_idx, issue_gather):
        gather_wait(k, b)
        if issue_gather:
            idx_wait(k + 1, 1 - b)
            gather(k + 1, 1 - b)
        scatter(k, b)
        if issue_idx:
            idx_load(k + 2, b)

    def body(i, carry):
        step(2 * i, 0, True, True)
        step(2 * i + 1, 1, True, True)
        return carry

    lax.fori_loop(0, (ec - 2) // 2, body, 0)
    step(ec - 2, 0, False, True)
    step(ec - 1, 1, False, False)

    plsc.subcore_barrier()
    pltpu.sync_copy(agg_sp.at[pl.ds(rbase, _ROWS_PER_TILE)],
                    agg_out.at[cid, pl.ds(rbase, _ROWS_PER_TILE)])
    if with_count:
        pltpu.sync_copy(cnt_sp.at[pl.ds(rbase, _ROWS_PER_TILE)],
                        cnt_out.at[cid, pl.ds(rbase, _ROWS_PER_TILE)])


@functools.lru_cache(maxsize=None)
def _agg_count_call():
    return functools.partial(
        pl.kernel,
        mesh=_mesh(),
        out_type=[jax.ShapeDtypeStruct((2, _NP, _D), jnp.float32),
                  jax.ShapeDtypeStruct((2, _NP), jnp.float32)],
        scratch_types=[
            pltpu.VMEM((2, _CH), jnp.int32),
            pltpu.VMEM((2, _CH), jnp.int32),
            pltpu.VMEM((2, _CH, _D), jnp.float32),
            pltpu.VMEM((_CH,), jnp.float32),
            pltpu.VMEM_SHARED((_NP, _D), jnp.float32),
            pltpu.VMEM_SHARED((_NP,), jnp.float32),
        ] + [pltpu.SemaphoreType.DMA] * 4,
    )(functools.partial(_agg_body, True))


@functools.lru_cache(maxsize=None)
def _agg_call():
    return functools.partial(
        pl.kernel,
        mesh=_mesh(),
        out_type=[jax.ShapeDtypeStruct((2, _NP, _D), jnp.float32)],
        scratch_types=[
            pltpu.VMEM((2, _CH), jnp.int32),
            pltpu.VMEM((2, _CH), jnp.int32),
            pltpu.VMEM((2, _CH, _D), jnp.float32),
            pltpu.VMEM_SHARED((_NP, _D), jnp.float32),
        ] + [pltpu.SemaphoreType.DMA] * 4,
    )(functools.partial(_agg_body, False))


_R = 2048


def _tc_layer_body(relu, want_norm, agg_ref, cnt_ref, h_ref, wl_ref, wr_ref,
                   b_ref, z_ref, *nz_ref):
    a = agg_ref[0] + agg_ref[1]
    c = cnt_ref[0] + cnt_ref[1]
    mean = a * (1.0 / jnp.maximum(c, 1.0))
    dn = (((1,), (1,)), ((), ()))
    z = (lax.dot_general(mean, wl_ref[...], dn,
                         precision=lax.Precision.HIGHEST,
                         preferred_element_type=jnp.float32)
         + lax.dot_general(h_ref[...], wr_ref[...], dn,
                           precision=lax.Precision.HIGHEST,
                           preferred_element_type=jnp.float32)
         + b_ref[...])
    if relu:
        z = jnp.maximum(z, 0.0)
    z_ref[...] = z
    if want_norm:
        nz_ref[0][...] = jnp.sqrt(jnp.sum(z * z, axis=1, keepdims=True))


def _tc_layer(aggp, cntp, hin, Wl, Wr, b, relu, want_norm):
    grid = (_NP // _R,)
    out_shape = [jax.ShapeDtypeStruct((_NP, _D), jnp.float32)]
    out_specs = [pl.BlockSpec((_R, _D), lambda i: (i, 0))]
    if want_norm:
        out_shape.append(jax.ShapeDtypeStruct((_NP, 1), jnp.float32))
        out_specs.append(pl.BlockSpec((_R, 1), lambda i: (i, 0)))
    return pl.pallas_call(
        functools.partial(_tc_layer_body, relu, want_norm),
        grid=grid,
        in_specs=[
            pl.BlockSpec((2, _R, _D), lambda i: (0, i, 0)),
            pl.BlockSpec((2, _R, 1), lambda i: (0, i, 0)),
            pl.BlockSpec((_R, _D), lambda i: (i, 0)),
            pl.BlockSpec((_D, _D), lambda i: (0, 0)),
            pl.BlockSpec((_D, _D), lambda i: (0, 0)),
            pl.BlockSpec((1, _D), lambda i: (0, 0)),
        ],
        out_specs=out_specs,
        out_shape=out_shape,
    )(aggp, cntp, hin, Wl, Wr, b)


def _gather_pairs_body(z_hbm, ia_hbm, ib_hbm, za_out, zb_out,
                       aidx, bidx, za, zb, *sems):
    semi = sems[0:2]
    semga = sems[2:4]
    semgb = sems[4:6]
    semwa = sems[6:8]
    semwb = sems[8:10]
    cid = lax.axis_index("c")
    sid = lax.axis_index("s")
    fast = cid == _FAST_CID
    dc = jnp.where(fast, _DCF, _DCS)
    ibase = jnp.where(fast, sid * _DCF, 16 * _DCF + sid * _DCS)
    obase = ibase * _CH

    def idx_load(k, b):
        pltpu.async_copy(ia_hbm.at[ibase + k], aidx.at[b], semi[b])
        pltpu.async_copy(ib_hbm.at[ibase + k], bidx.at[b], semi[b])

    def idx_wait(k, b):
        pltpu.make_async_copy(ia_hbm.at[ibase + k], aidx.at[b], semi[b]).wait()
        pltpu.make_async_copy(ib_hbm.at[ibase + k], bidx.at[b], semi[b]).wait()

    def gather(k, b):
        pltpu.async_copy(z_hbm.at[aidx.at[b]], za.at[b], semga[b])
        pltpu.async_copy(z_hbm.at[bidx.at[b]], zb.at[b], semgb[b])

    def gather_wait(k, b):
        pltpu.make_async_copy(z_hbm.at[aidx.at[b]], za.at[b], semga[b]).wait()
        pltpu.make_async_copy(z_hbm.at[bidx.at[b]], zb.at[b], semgb[b]).wait()

    def write(k, b):
        o = obase + k * _CH
        pltpu.async_copy(za.at[b], za_out.at[pl.ds(o, _CH)], semwa[b])
        pltpu.async_copy(zb.at[b], zb_out.at[pl.ds(o, _CH)], semwb[b])

    def write_wait(k, b):
        o = obase + k * _CH
        pltpu.make_async_copy(za.at[b], za_out.at[pl.ds(o, _CH)], semwa[b]).wait()
        pltpu.make_async_copy(zb.at[b], zb_out.at[pl.ds(o, _CH)], semwb[b]).wait()

    def step(k, b, wait_prev, issue_idx, issue_gather):
        gather_wait(k, b)
        write(k, b)
        if wait_prev:
            write_wait(k - 1, 1 - b)
        if issue_gather:
            idx_wait(k + 1, 1 - b)
            gather(k + 1, 1 - b)
        if issue_idx:
            idx_load(k + 2, b)

    idx_load(0, 0)
    idx_load(1, 1)
    idx_wait(0, 0)
    gather(0, 0)
    step(0, 0, False, True, True)

    def body(i, carry):
        k = 2 * i + 1
        step(k, 1, True, True, True)
        step(k + 1, 0, True, True, True)
        return carry

    lax.fori_loop(0, (dc - 3) // 2, body, 0)
    step(dc - 2, 1, True, False, True)
    step(dc - 1, 0, True, False, False)
    write_wait(dc - 1, 0)


@functools.lru_cache(maxsize=None)
def _gather_pairs_call():
    return functools.partial(
        pl.kernel,
        mesh=_mesh(),
        out_type=[jax.ShapeDtypeStruct((_ELP, _D), jnp.float32),
                  jax.ShapeDtypeStruct((_ELP, _D), jnp.float32)],
        scratch_types=[
            pltpu.VMEM((2, _CH), jnp.int32),
            pltpu.VMEM((2, _CH), jnp.int32),
            pltpu.VMEM((2, _CH, _D), jnp.float32),
            pltpu.VMEM((2, _CH, _D), jnp.float32),
        ] + [pltpu.SemaphoreType.DMA] * 10,
    )(_gather_pairs_body)


_RD = 2048


def _cosine_body(za_ref, zb_ref, o_ref):
    za = za_ref[...]
    zb = zb_ref[...]
    num = jnp.sum(za * zb, axis=1, keepdims=True)
    sa = jnp.sum(za * za, axis=1, keepdims=True)
    sb = jnp.sum(zb * zb, axis=1, keepdims=True)
    den = jnp.maximum(jnp.sqrt(sa) * jnp.sqrt(sb), 1e-8)
    o_ref[...] = num / den


def _cosine(za, zb):
    return pl.pallas_call(
        _cosine_body,
        grid=(_ELP // _RD,),
        in_specs=[pl.BlockSpec((_RD, _D), lambda i: (i, 0)),
                  pl.BlockSpec((_RD, _D), lambda i: (i, 0))],
        out_specs=pl.BlockSpec((_RD, 1), lambda i: (i, 0)),
        out_shape=jax.ShapeDtypeStruct((_ELP, 1), jnp.float32),
    )(za, zb)


def kernel(x, edge_index, edge_label_index, W1l, W1r, b1, W2l, W2r, b2):
    src = edge_index[0]
    dst = edge_index[1]
    srcp = jnp.concatenate([src, jnp.zeros((_EP - _E,), jnp.int32)])
    srcp = srcp.reshape(_EP // _CH, _CH)
    dstp = jnp.concatenate([dst, jnp.full((_EP - _E,), _N, jnp.int32)])
    dstp = dstp.reshape(_EP // _CH, _CH)
    xp = jnp.concatenate([x, jnp.zeros((_NP - _N, _D), jnp.float32)], axis=0)
    zr = jnp.zeros((_ROWS_PER_TILE, _D), jnp.float32)
    zc = jnp.zeros((_ROWS_PER_TILE,), jnp.float32)

    agg1, cnt = _agg_count_call()(xp, srcp, dstp, zr, zc)
    cnt3 = cnt.reshape(2, _NP, 1)
    h = _tc_layer(agg1, cnt3, xp, W1l, W1r, b1.reshape(1, _D),
                  relu=True, want_norm=False)[0]
    (agg2,) = _agg_call()(h, srcp, dstp, zr)
    (z,) = _tc_layer(agg2, cnt3, h, W2l, W2r, b2.reshape(1, _D),
                     relu=False, want_norm=False)

    ea = jnp.concatenate([edge_label_index[0],
                          jnp.zeros((_ELP - _EL,), jnp.int32)])
    ea = ea.reshape(_ELP // _CH, _CH)
    eb = jnp.concatenate([edge_label_index[1],
                          jnp.zeros((_ELP - _EL,), jnp.int32)])
    eb = eb.reshape(_ELP // _CH, _CH)
    za, zb = _gather_pairs_call()(z, ea, eb)
    out = _cosine(za, zb)
    return out.reshape(_ELP)[:_EL]
